# baseline (speedup 1.0000x reference)
import jax
import jax.numpy as jnp
from jax import lax
from jax.experimental import pallas as pl
from jax.experimental.pallas import tpu as pltpu
from jax.experimental.pallas import tpu_sc as plsc

NUM_BUCKETS = 1000000
HALF_BUCKETS = NUM_BUCKETS // 2
DIM = 64
B_ROWS = 4
SEQ = 8192
TOTAL = B_ROWS * SEQ

_info = plsc.get_sparse_core_info()
NC, NS, L = _info.num_cores, _info.num_subcores, _info.num_lanes
NW = NC * NS
B_PER_W = TOTAL // NW
PASS_ROWS = 256
N_PASS = B_PER_W // PASS_ROWS
GCHUNK = 128
N_G = PASS_ROWS // GCHUNK


def _sc_kernel(
    ids_hbm, t0_hbm, t1_hbm, out_hbm,
    ext_v, idx0_v, idx1_v, fsel_v, rows0_v, rows1_v, stage_v, sem,
):
    wid = lax.axis_index("s") * NC + lax.axis_index("c")
    base = wid * B_PER_W

    pltpu.sync_copy(ids_hbm.at[pl.ds(base, B_PER_W + 8)], ext_v)

    lane = lax.iota(jnp.int32, 16)

    def make_hash_step(p):
        def hash_step(s, _):
            i0 = s * 16
            cur = ext_v[pl.ds(i0 + 8, 16)]
            prev = ext_v[pl.ds(i0 + 7, 16)]
            prev = prev * jnp.minimum((base + i0 + lane) & (SEQ - 1), 1)
            h = (prev * 31 + cur) % NUM_BUCKETS
            o = i0 - p * PASS_ROWS
            idx0_v[pl.ds(o, 16)] = jnp.minimum(h, HALF_BUCKETS - 1)
            idx1_v[pl.ds(o, 16)] = jnp.clip(
                h - HALF_BUCKETS, 0, HALF_BUCKETS - 1
            )
            fsel_v[pl.ds(o, 16)] = jnp.minimum(
                jnp.maximum(HALF_BUCKETS - h, 0), 1
            ).astype(jnp.float32)
            return 0

        return hash_step

    for p in range(N_PASS):
        lax.fori_loop(
            p * (PASS_ROWS // 16),
            (p + 1) * (PASS_ROWS // 16),
            make_hash_step(p),
            0,
            unroll=8,
        )

        copies = []
        for g in range(N_G):
            sl = pl.ds(g * GCHUNK, GCHUNK)
            copies.append(
                pltpu.async_copy(t0_hbm.at[idx0_v.at[sl]], rows0_v.at[sl], sem)
            )
            copies.append(
                pltpu.async_copy(t1_hbm.at[idx1_v.at[sl]], rows1_v.at[sl], sem)
            )
        for c in copies:
            c.wait()

        def select_row(j, _):
            fm = plsc.load_gather(fsel_v, [lane * 0 + j])
            for c in range(DIM // 16):
                a = rows0_v[j, pl.ds(c * 16, 16)]
                b = rows1_v[j, pl.ds(c * 16, 16)]
                stage_v[j, pl.ds(c * 16, 16)] = b + (a - b) * fm
            return 0

        lax.fori_loop(0, PASS_ROWS, select_row, 0, unroll=4)

        pltpu.sync_copy(
            stage_v, out_hbm.at[pl.ds(base + p * PASS_ROWS, PASS_ROWS)]
        )


@jax.jit
def kernel(input_ids, emb_weight):
    ids_flat = input_ids.reshape(-1).astype(jnp.int32)
    ids_pad = jnp.concatenate([jnp.zeros((8,), jnp.int32), ids_flat])
    t0 = emb_weight[:HALF_BUCKETS]
    t1 = emb_weight[HALF_BUCKETS:]

    mesh = plsc.VectorSubcoreMesh(core_axis_name="c", subcore_axis_name="s")
    out = pl.kernel(
        _sc_kernel,
        mesh=mesh,
        out_type=jax.ShapeDtypeStruct((TOTAL, 2 * DIM), jnp.float32),
        scratch_types=[
            pltpu.VMEM((B_PER_W + 8,), jnp.int32),
            pltpu.VMEM((PASS_ROWS,), jnp.int32),
            pltpu.VMEM((PASS_ROWS,), jnp.int32),
            pltpu.VMEM((PASS_ROWS,), jnp.float32),
            pltpu.VMEM((PASS_ROWS, DIM), jnp.float32),
            pltpu.VMEM((PASS_ROWS, DIM), jnp.float32),
            pltpu.VMEM((PASS_ROWS, 2 * DIM), jnp.float32),
            pltpu.SemaphoreType.DMA,
        ],
        compiler_params=pltpu.CompilerParams(
            use_tc_tiling_on_sc=False, needs_layout_passes=False
        ),
    )(ids_pad, t0, t1)
    return out[:, :DIM].reshape(B_ROWS, SEQ, DIM)


# speedup vs baseline: 2.4227x; 2.4227x over previous
import jax
import jax.numpy as jnp
from jax import lax
from jax.experimental import pallas as pl
from jax.experimental.pallas import tpu as pltpu
from jax.experimental.pallas import tpu_sc as plsc

NUM_BUCKETS = 1000000
DIM = 64
B_ROWS = 4
SEQ = 8192
TOTAL = B_ROWS * SEQ

_info = plsc.get_sparse_core_info()
NC, NS, L = _info.num_cores, _info.num_subcores, _info.num_lanes
NW = NC * NS
B_PER_W = TOTAL // NW
N_VEC = B_PER_W // 16

def _lane(v, j):
    return v[j]


NSEM = 4
DIRECT = 384
STAGED = B_PER_W - DIRECT


def _sc_kernel(ids_hbm, table_hbm, out_hbm, ext_v, rows_v, sems):
    wid = lax.axis_index("s") * NC + lax.axis_index("c")
    base = wid * B_PER_W

    pltpu.sync_copy(ids_hbm.at[pl.ds(base, B_PER_W + 8)], ext_v)

    lane = lax.iota(jnp.int32, 16)

    def hash_vec(i0):
        cur = ext_v[pl.ds(i0 + 8, 16)]
        prev = ext_v[pl.ds(i0 + 7, 16)]
        prev = prev * jnp.minimum((base + i0 + lane) & (SEQ - 1), 1)
        return (prev * 31 + cur) % NUM_BUCKETS

    def group_direct(g, _):
        i0 = g * 16
        h = hash_vec(i0)
        for j in range(16):
            pltpu.async_copy(
                table_hbm.at[pl.ds(h[j], 1)],
                out_hbm.at[pl.ds(base + i0 + j, 1)],
                sems.at[NSEM + (j % NSEM)],
            )
        return 0

    lax.fori_loop(0, DIRECT // 16, group_direct, 0, unroll=2)

    def group_staged(g, _):
        i0 = g * 16
        h = hash_vec(i0)
        for j in range(16):
            pltpu.async_copy(
                table_hbm.at[pl.ds(h[j], 1)],
                rows_v.at[pl.ds(i0 - DIRECT + j, 1)],
                sems.at[j % NSEM],
            )
        return 0

    lax.fori_loop(DIRECT // 16, N_VEC, group_staged, 0, unroll=2)

    for q in range(NSEM):
        n = STAGED // NSEM
        pltpu.make_async_copy(
            table_hbm.at[pl.ds(0, n)],
            rows_v.at[pl.ds(q * n, n)],
            sems.at[q],
        ).wait()
    pltpu.sync_copy(rows_v, out_hbm.at[pl.ds(base + DIRECT, STAGED)])

    for q in range(NSEM):
        n = DIRECT // NSEM
        pltpu.make_async_copy(
            table_hbm.at[pl.ds(0, n)],
            out_hbm.at[pl.ds(base + q * n, n)],
            sems.at[NSEM + q],
        ).wait()


@jax.jit
def kernel(input_ids, emb_weight):
    ids_flat = input_ids.reshape(-1).astype(jnp.int32)
    ids_pad = jnp.concatenate([jnp.zeros((8,), jnp.int32), ids_flat])

    mesh = plsc.VectorSubcoreMesh(core_axis_name="c", subcore_axis_name="s")
    out = pl.kernel(
        _sc_kernel,
        mesh=mesh,
        out_type=jax.ShapeDtypeStruct((TOTAL, DIM), jnp.float32),
        scratch_types=[
            pltpu.VMEM((B_PER_W + 8,), jnp.int32),
            pltpu.VMEM((STAGED, DIM), jnp.float32),
            pltpu.SemaphoreType.DMA((2 * NSEM,)),
        ],
        compiler_params=pltpu.CompilerParams(
            use_tc_tiling_on_sc=True, needs_layout_passes=False
        ),
    )(ids_pad, emb_weight)
    return out.reshape(B_ROWS, SEQ, DIM)


# speedup vs baseline: 3.5545x; 1.4672x over previous
import jax
import jax.numpy as jnp
from jax import lax
from jax.experimental import pallas as pl
from jax.experimental.pallas import tpu as pltpu
from jax.experimental.pallas import tpu_sc as plsc

NUM_BUCKETS = 1000000
DIM = 64
B_ROWS = 4
SEQ = 8192
TOTAL = B_ROWS * SEQ

_info = plsc.get_sparse_core_info()
NC, NS, L = _info.num_cores, _info.num_subcores, _info.num_lanes
NW = NC * NS
B_PER_W = TOTAL // NW
N_VEC = B_PER_W // 16

def _lane(v, j):
    return v[j]


NSEM = 4
HALF = B_PER_W // 2


def _sc_kernel(ids_hbm, table_hbm, out_hbm, ext_v, rows_v, sems):
    wid = lax.axis_index("s") * NC + lax.axis_index("c")
    base = wid * B_PER_W

    pltpu.sync_copy(ids_hbm.at[pl.ds(base, B_PER_W + 8)], ext_v)

    lane = lax.iota(jnp.int32, 16)

    def make_group(p):
        def group(g, _):
            i0 = g * 16
            cur = ext_v[pl.ds(i0 + 8, 16)]
            prev = ext_v[pl.ds(i0 + 7, 16)]
            prev = prev * jnp.minimum((base + i0 + lane) & (SEQ - 1), 1)
            h = (prev * 31 + cur) % NUM_BUCKETS
            for j in range(16):
                r = _lane(h, j)
                pltpu.async_copy(
                    table_hbm.at[pl.ds(r, 1)],
                    rows_v.at[pl.ds(i0 - p * HALF + j, 1)],
                    sems.at[j % NSEM],
                )
            return 0

        return group

    for p in range(2):
        lax.fori_loop(
            p * (HALF // 16), (p + 1) * (HALF // 16), make_group(p), 0, unroll=2
        )
        for q in range(NSEM):
            pltpu.make_async_copy(
                table_hbm.at[pl.ds(0, HALF // NSEM)],
                rows_v.at[pl.ds(q * (HALF // NSEM), HALF // NSEM)],
                sems.at[q],
            ).wait()
        pltpu.sync_copy(rows_v, out_hbm.at[pl.ds(base + p * HALF, HALF)])


@jax.jit
def kernel(input_ids, emb_weight):
    ids_flat = input_ids.reshape(-1).astype(jnp.int32)
    ids_pad = jnp.concatenate([jnp.zeros((8,), jnp.int32), ids_flat])

    mesh = plsc.VectorSubcoreMesh(core_axis_name="c", subcore_axis_name="s")
    out = pl.kernel(
        _sc_kernel,
        mesh=mesh,
        out_type=jax.ShapeDtypeStruct((TOTAL, DIM), jnp.float32),
        scratch_types=[
            pltpu.VMEM((B_PER_W + 8,), jnp.int32),
            pltpu.VMEM((HALF, DIM), jnp.float32),
            pltpu.SemaphoreType.DMA((NSEM,)),
        ],
        compiler_params=pltpu.CompilerParams(
            use_tc_tiling_on_sc=True, needs_layout_passes=False
        ),
    )(ids_pad, emb_weight)
    return out.reshape(B_ROWS, SEQ, DIM)
